# Initial kernel scaffold; baseline (speedup 1.0000x reference)
import jax
import jax.numpy as jnp
from jax.experimental import pallas as pl


def kernel(hidden_states, router_weight, router_bias, gate_up_proj, gate_up_proj_bias, down_proj, down_proj_bias):
    raise NotImplementedError("write your pallas kernel here")


# speedup vs baseline: 20.4718x; 20.4718x over previous
import functools
import jax
import jax.numpy as jnp
from jax.experimental import pallas as pl
from jax.experimental.pallas import tpu as pltpu

E = 8
ALPHA = 1.702
LIMIT = 7.0


def _router_body(x_ref, wT_ref, b_ref, rw_ref):
    x = x_ref[...]
    logits = jnp.dot(x, wT_ref[...], preferred_element_type=jnp.float32)
    logits = logits + b_ref[...]
    t = logits.shape[0]
    iota = jax.lax.broadcasted_iota(jnp.int32, (t, E), 1)
    m1 = jnp.max(logits, axis=1, keepdims=True)
    i1 = jnp.min(jnp.where(logits == m1, iota, E), axis=1, keepdims=True)
    masked = jnp.where(iota == i1, -jnp.inf, logits)
    m2 = jnp.max(masked, axis=1, keepdims=True)
    i2 = jnp.min(jnp.where(masked == m2, iota, E), axis=1, keepdims=True)
    r = jnp.exp(m2 - m1)
    w1 = 1.0 / (1.0 + r)
    w2 = 1.0 - w1
    rw_ref[...] = jnp.where(iota == i1, w1, 0.0) + jnp.where(iota == i2, w2, 0.0)


def _ffn_body(rw_ref, x_ref, wg_ref, wu_ref, wd_ref, bg_ref, bu_ref, bd_ref,
              o_ref):
    e = pl.program_id(0)
    i = pl.program_id(1)

    @pl.when((e == 0) & (i == 0))
    def _():
        o_ref[...] = jnp.zeros_like(o_ref)

    x = x_ref[...]
    gate = jnp.dot(x, wg_ref[0], preferred_element_type=jnp.float32) + bg_ref[0]
    up = jnp.dot(x, wu_ref[0], preferred_element_type=jnp.float32) + bu_ref[0]
    gate = jnp.minimum(gate, LIMIT)
    up = jnp.clip(up, -LIMIT, LIMIT)
    glu = gate * (1.0 / (1.0 + jnp.exp(-ALPHA * gate)))
    act = (up + 1.0) * glu
    y = jnp.dot(act, wd_ref[0], preferred_element_type=jnp.float32)
    first = (i == 0).astype(jnp.float32)
    y = y + first * bd_ref[0]
    rw = rw_ref[...]
    lane = jax.lax.broadcasted_iota(jnp.int32, rw.shape, 1)
    rwcol = jnp.sum(jnp.where(lane == e, rw, 0.0), axis=1, keepdims=True)
    o_ref[...] += y * rwcol


def kernel(hidden_states, router_weight, router_bias, gate_up_proj,
           gate_up_proj_bias, down_proj, down_proj_bias):
    b, s, h = hidden_states.shape
    t = b * s
    i_dim = down_proj.shape[1]
    x2 = hidden_states.reshape(t, h)

    rw = pl.pallas_call(
        _router_body,
        out_shape=jax.ShapeDtypeStruct((t, E), jnp.float32),
    )(x2, router_weight.T, router_bias.reshape(1, E))

    gu = gate_up_proj.reshape(E, h, i_dim, 2)
    wg = gu[..., 0]
    wu = gu[..., 1]
    bgu = gate_up_proj_bias.reshape(E, 1, i_dim, 2)
    bg = bgu[..., 0]
    bu = bgu[..., 1]
    bd = down_proj_bias.reshape(E, 1, h)

    bi = 512
    ni = i_dim // bi
    out = pl.pallas_call(
        _ffn_body,
        grid=(E, ni),
        in_specs=[
            pl.BlockSpec((t, E), lambda e, i: (0, 0)),
            pl.BlockSpec((t, h), lambda e, i: (0, 0)),
            pl.BlockSpec((1, h, bi), lambda e, i: (e, 0, i)),
            pl.BlockSpec((1, h, bi), lambda e, i: (e, 0, i)),
            pl.BlockSpec((1, bi, h), lambda e, i: (e, i, 0)),
            pl.BlockSpec((1, 1, bi), lambda e, i: (e, 0, i)),
            pl.BlockSpec((1, 1, bi), lambda e, i: (e, 0, i)),
            pl.BlockSpec((1, 1, h), lambda e, i: (e, 0, 0)),
        ],
        out_specs=pl.BlockSpec((t, h), lambda e, i: (0, 0)),
        out_shape=jax.ShapeDtypeStruct((t, h), jnp.float32),
        compiler_params=pltpu.CompilerParams(
            dimension_semantics=("arbitrary", "arbitrary")),
    )(rw, x2, wg, wu, down_proj, bg, bu, bd)

    return out.reshape(b, s, h)
